# baseline (speedup 1.0000x reference)
import jax
import jax.numpy as jnp
import numpy as np
from jax.experimental import pallas as pl
from jax.experimental.pallas import tpu as pltpu

_G = 32
_TLEN = 16
_NN = 16
_D2 = 16
_HID = 8
_EPS = 1e-5
_DECAY = 0.7
_NEG = 0.01
_R = 256


def _leaky(x):
    return jnp.maximum(x, _NEG * x)


def _bdot(a, b):
    return jnp.dot(a.astype(jnp.bfloat16), b.astype(jnp.bfloat16),
                   preferred_element_type=jnp.float32)


def _body(xu_ref, pe_ref, m1_ref, s1v_ref, b1v_ref, m2_ref, s2v_ref, b2v_ref,
          l2_ref, l2b_ref, slv_ref, blb_ref,
          g1t_ref, g1b_ref, g2t_ref, g2b_ref,
          sb1_ref, bb1_ref, sb2_ref, bb2_ref,
          t1t_ref, t1b_ref, m1s_ref, m1b_ref,
          t2t_ref, t2b_ref, m2s_ref, m2b_ref,
          mx1_ref, iw1_ref,
          mx2_ref, iw2_ref,
          wpq_ref, fb1_ref,
          f2t_ref, fb2_ref, f3t_ref, fb3_ref, f4_ref, fb4_ref,
          out_ref):
    f32 = jnp.float32
    x = xu_ref[...]
    h = jnp.maximum(_bdot(x, m1_ref[...]) * s1v_ref[...] + b1v_ref[...], 0.0)
    h = jnp.maximum(_bdot(h, m2_ref[...]) * s2v_ref[...] + b2v_ref[...], 0.0)
    a4 = ((_bdot(h, l2_ref[...]) + l2b_ref[...]) * slv_ref[...] + blb_ref[...]
          + pe_ref[...])

    nf1 = _bdot(a4, g1t_ref[...]) + g1b_ref[...]
    nf2 = _bdot(a4, g2t_ref[...]) + g2b_ref[...]
    xb1 = a4 * sb1_ref[...] + bb1_ref[...]
    xb2 = a4 * sb2_ref[...] + bb2_ref[...]

    dn = (((1,), (1,)), ((), ()))
    dn0 = (((0,), (0,)), ((), ()))
    bf16 = jnp.bfloat16
    fs = []
    for b in range(_G):
        r0 = b * _R
        nf1b = jax.lax.slice(nf1, (r0, 0), (r0 + _R, _D2)).astype(bf16)
        nf2b = jax.lax.slice(nf2, (r0, 0), (r0 + _R, _D2)).astype(bf16)
        xb1b = jax.lax.slice(xb1, (r0, 0), (r0 + _R, _D2))
        xb2b = jax.lax.slice(xb2, (r0, 0), (r0 + _R, _D2))
        g1 = jax.lax.dot_general(nf1b, nf1b, dn, preferred_element_type=f32)
        g2 = jax.lax.dot_general(nf2b, nf2b, dn, preferred_element_type=f32)

        e1 = jnp.exp(_leaky(g1))
        s1 = jnp.sum(e1 * iw1_ref[...], axis=-1, keepdims=True)
        xb1c = xb1b.astype(jnp.bfloat16)
        a1 = (e1 / s1) * mx1_ref[...]
        h1 = _bdot(a1, xb1c) + xb1c.astype(f32)
        h1 = _bdot(h1, t1t_ref[...]) + t1b_ref[...]
        v1 = _leaky(h1 * m1s_ref[...] + m1b_ref[...])

        e2f = jnp.exp(_leaky(g2))
        e2 = jnp.concatenate(
            [jax.lax.slice(e2f, (64 * j, 0), (64 * j + 128, _R))
             for j in range(3)], axis=0)
        s2 = jnp.sum(e2 * iw2_ref[...], axis=-1, keepdims=True)
        xb2c = xb2b.astype(jnp.bfloat16)
        z2s = jnp.concatenate(
            [jax.lax.slice(xb2c, (64 * j, 0), (64 * j + 128, _D2))
             for j in range(3)], axis=0)
        a2 = (e2 / s2) * mx2_ref[...]
        h2 = _bdot(a2, xb2c) + z2s.astype(f32)
        h2 = _bdot(h2, t2t_ref[...]) + t2b_ref[...]
        v2 = _leaky(h2 * m2s_ref[...] + m2b_ref[...])

        pieces = []
        for j in range(4):
            acc = jax.lax.slice(v1, (64 * j, 0), (64 * j + 16, _HID))
            for k in range(1, 4):
                acc = acc + jax.lax.slice(
                    v1, (64 * j + 16 * k, 0), (64 * j + 16 * k + 16, _HID))
            pieces.append(acc * 0.25)
        for j in range(3):
            acc = jax.lax.slice(v2, (128 * j, 0), (128 * j + 16, _HID))
            for k in range(1, 8):
                acc = acc + jax.lax.slice(
                    v2, (128 * j + 16 * k, 0), (128 * j + 16 * k + 16, _HID))
            pieces.append(acc * 0.125)
        hcat = jnp.concatenate(pieces, axis=0)

        c = _bdot(hcat.T, wpq_ref[...])
        f = jax.lax.slice(c, (0, 0), (1, 16))
        for hh in range(1, _HID):
            f = f + jax.lax.slice(c, (hh, 16 * hh), (hh + 1, 16 * hh + 16))
        fs.append(f)

    f = jnp.concatenate(fs, axis=0)
    f = jnp.maximum(f + fb1_ref[...], 0.0)
    f = jnp.maximum(_bdot(f, f2t_ref[...]) + fb2_ref[...], 0.0)
    f = jnp.maximum(_bdot(f, f3t_ref[...]) + fb3_ref[...], 0.0)
    y = (jnp.sum(f.astype(bf16).astype(f32)
                 * f4_ref[...].astype(bf16).astype(f32), axis=-1)
         + fb4_ref[0, 0])
    out_ref[...] = jnp.broadcast_to(y[:, None, None], (_G, 8, 128))


@jax.jit
def kernel(X, params):
    p = params
    f32 = jnp.float32
    bs = X.shape[0]

    xu = jnp.transpose(X.reshape(bs, _TLEN, 16, _NN), (0, 1, 3, 2))
    xu = xu.reshape(bs * _TLEN * _NN, 16)

    ti = jnp.arange(16)[:, None] - jnp.arange(16)[None, :]
    bands = jnp.stack([(ti == k - 1).astype(f32) for k in range(3)])
    m1 = jnp.einsum('ck,ktu->ctu', p['conv1_w'][:, 0, :], bands)
    m1 = jnp.transpose(m1, (1, 0, 2)).reshape(16, 128)
    s1v = jnp.repeat(p['bn_c1_g'] / jnp.sqrt(1.0 + _EPS), 16)[None, :]
    b1v = jnp.repeat(p['bn_c1_b'], 16)[None, :]
    m2 = jnp.einsum('oik,ktu->itou', p['conv2_w'], bands).reshape(128, 128)
    s2v = jnp.repeat(p['bn_c2_g'] / jnp.sqrt(1.0 + _EPS), 16)[None, :]
    b2v = jnp.repeat(p['bn_c2_b'], 16)[None, :]

    l2 = p['lin2_w'].T
    l2b = p['lin2_b'][None, :]
    slv = (p['bn2_g'] / jnp.sqrt(1.0 + _EPS))[None, :]
    blb = p['bn2_b'][None, :]

    pos = jnp.arange(_TLEN, dtype=f32)[:, None]
    div = jnp.exp(jnp.arange(0, _D2, 2, dtype=f32) * (-np.log(10000.0) / _D2))
    pe = jnp.zeros((_TLEN, _D2), f32)
    pe = pe.at[:, 0::2].set(jnp.sin(pos * div))
    pe = pe.at[:, 1::2].set(jnp.cos(pos * div))
    pe_exp = jnp.tile(jnp.repeat(pe, _NN, axis=0), (_G, 1))

    def bn_pair(g, b):
        return (g / jnp.sqrt(1.0 + _EPS))[None, :], b[None, :]

    sb1, bb1 = bn_pair(p['bnb1_g'], p['bnb1_b'])
    sb2, bb2 = bn_pair(p['bnb2_g'], p['bnb2_b'])
    m1s, m1b = bn_pair(p['bnm1_g'], p['bnm1_b'])
    m2s, m2b = bn_pair(p['bnm2_g'], p['bnm2_b'])

    r = jnp.arange(_R)
    win1 = r // 64
    inw1 = (win1[:, None] == win1[None, :]).astype(f32)
    eye = jnp.eye(_R, dtype=f32)
    pat = r // 16
    dec = _DECAY ** jnp.abs(pat[:, None] - pat[None, :]).astype(f32)
    mx1 = dec * inw1 - eye
    iw1 = inw1 - eye

    RR = jnp.arange(384)
    j2 = RR // 128
    rl = RR % 128
    col = jnp.arange(_R)[None, :]
    diagcol = (64 * j2 + rl)[:, None]
    inw2 = ((col >= (64 * j2)[:, None]) & (col < (64 * j2 + 128)[:, None]))
    pat_c = (col - (64 * j2)[:, None]) // 16
    dec2 = _DECAY ** jnp.abs((rl // 16)[:, None] - pat_c).astype(f32)
    ond = (col == diagcol)
    mx2 = jnp.where(inw2 & ~ond, dec2, 0.0).astype(f32)
    iw2 = jnp.where(inw2 & ~ond, 1.0, 0.0).astype(f32)

    wpq = jnp.transpose(p['fc1_w'].reshape(_D2, 7 * _NN, _HID),
                        (1, 2, 0)).reshape(7 * _NN, 128)
    fb1 = p['fc1_b'][None, :]
    f2t, fb2 = p['fc2_w'].T, p['fc2_b'][None, :]
    f3t, fb3 = p['fc3_w'].T, p['fc3_b'][None, :]
    f4 = p['fc4_w'][0][None, :]
    fb4 = p['fc4_b'][None, :]

    full = lambda shp: pl.BlockSpec(shp, lambda b: tuple(0 for _ in shp))
    in_specs = [
        pl.BlockSpec((_G * _R, 16), lambda b: (b, 0)),
        full((_G * _R, 16)),
        full((16, 128)), full((1, 128)), full((1, 128)),
        full((128, 128)), full((1, 128)), full((1, 128)),
        full((128, 16)), full((1, 16)), full((1, 16)), full((1, 16)),
        full((16, 16)), full((1, 16)),
        full((16, 16)), full((1, 16)),
        full((1, 16)), full((1, 16)),
        full((1, 16)), full((1, 16)),
        full((16, 8)), full((1, 8)),
        full((1, 8)), full((1, 8)),
        full((16, 8)), full((1, 8)),
        full((1, 8)), full((1, 8)),
        full((_R, _R)), full((_R, _R)),
        full((384, _R)), full((384, _R)),
        full((7 * _NN, 128)), full((1, 16)),
        full((16, 16)), full((1, 16)),
        full((16, 8)), full((1, 8)),
        full((1, 8)), full((1, 1)),
    ]
    out = pl.pallas_call(
        _body,
        grid=(bs // _G,),
        in_specs=in_specs,
        out_specs=pl.BlockSpec((_G, 8, 128), lambda b: (b, 0, 0)),
        out_shape=jax.ShapeDtypeStruct((bs, 8, 128), f32),
        compiler_params=pltpu.CompilerParams(
            dimension_semantics=("arbitrary",),
        ),
    )(xu, pe_exp, m1, s1v, b1v, m2, s2v, b2v, l2, l2b, slv, blb,
      p['g1_w'].T, p['g1_b'][None, :],
      p['g2_w'].T, p['g2_b'][None, :],
      sb1, bb1, sb2, bb2,
      p['t1_w'].T, p['t1_b'][None, :], m1s, m1b,
      p['t2_w'].T, p['t2_b'][None, :], m2s, m2b,
      mx1, iw1, mx2, iw2,
      wpq, fb1, f2t, fb2, f3t, fb3, f4, fb4)
    return out[:, 0, :1]


# speedup vs baseline: 1.1954x; 1.1954x over previous
import jax
import jax.numpy as jnp
import numpy as np
from jax.experimental import pallas as pl
from jax.experimental.pallas import tpu as pltpu

_G = 16
_TLEN = 16
_NN = 16
_D2 = 16
_HID = 8
_EPS = 1e-5
_DECAY = 0.7
_NEG = 0.01
_R = 256


def _leaky(x):
    return jnp.maximum(x, _NEG * x)


def _bdot(a, b):
    return jnp.dot(a.astype(jnp.bfloat16), b.astype(jnp.bfloat16),
                   preferred_element_type=jnp.float32)


def _body(xu_ref, pe_ref, m1_ref, s1v_ref, b1v_ref, m2_ref, s2v_ref, b2v_ref,
          l2_ref, l2b_ref, slv_ref, blb_ref,
          g1t_ref, g1b_ref, g2t_ref, g2b_ref,
          sb1_ref, bb1_ref, sb2_ref, bb2_ref,
          t1t_ref, t1b_ref, m1s_ref, m1b_ref,
          t2t_ref, t2b_ref, m2s_ref, m2b_ref,
          mx1_ref, iw1_ref,
          mx2_ref, iw2_ref,
          wpq_ref, fb1_ref,
          f2t_ref, fb2_ref, f3t_ref, fb3_ref, f4_ref, fb4_ref,
          out_ref):
    f32 = jnp.float32
    x = xu_ref[...]
    h = jnp.maximum(_bdot(x, m1_ref[...]) * s1v_ref[...] + b1v_ref[...], 0.0)
    h = jnp.maximum(_bdot(h, m2_ref[...]) * s2v_ref[...] + b2v_ref[...], 0.0)
    a4 = ((_bdot(h, l2_ref[...]) + l2b_ref[...]) * slv_ref[...] + blb_ref[...]
          + pe_ref[...])

    nf1 = _bdot(a4, g1t_ref[...]) + g1b_ref[...]
    nf2 = _bdot(a4, g2t_ref[...]) + g2b_ref[...]
    xb1 = a4 * sb1_ref[...] + bb1_ref[...]
    xb2 = a4 * sb2_ref[...] + bb2_ref[...]

    dn = (((1,), (1,)), ((), ()))
    dn0 = (((0,), (0,)), ((), ()))
    bf16 = jnp.bfloat16
    fs = []
    for b in range(_G):
        r0 = b * _R
        nf1b = jax.lax.slice(nf1, (r0, 0), (r0 + _R, _D2)).astype(bf16)
        nf2b = jax.lax.slice(nf2, (r0, 0), (r0 + _R, _D2)).astype(bf16)
        xb1b = jax.lax.slice(xb1, (r0, 0), (r0 + _R, _D2))
        xb2b = jax.lax.slice(xb2, (r0, 0), (r0 + _R, _D2))
        g1 = jax.lax.dot_general(nf1b, nf1b, dn, preferred_element_type=f32)
        g2 = jax.lax.dot_general(nf2b, nf2b, dn, preferred_element_type=f32)

        e1 = jnp.exp(_leaky(g1))
        s1 = jnp.sum(e1 * iw1_ref[...], axis=-1, keepdims=True)
        xb1c = xb1b.astype(jnp.bfloat16)
        a1 = (e1 / s1) * mx1_ref[...]
        h1 = _bdot(a1, xb1c) + xb1c.astype(f32)
        h1 = _bdot(h1, t1t_ref[...]) + t1b_ref[...]
        v1 = _leaky(h1 * m1s_ref[...] + m1b_ref[...])

        e2f = jnp.exp(_leaky(g2))
        e2 = jnp.concatenate(
            [jax.lax.slice(e2f, (64 * j, 0), (64 * j + 128, _R))
             for j in range(3)], axis=0)
        s2 = jnp.sum(e2 * iw2_ref[...], axis=-1, keepdims=True)
        xb2c = xb2b.astype(jnp.bfloat16)
        z2s = jnp.concatenate(
            [jax.lax.slice(xb2c, (64 * j, 0), (64 * j + 128, _D2))
             for j in range(3)], axis=0)
        a2 = (e2 / s2) * mx2_ref[...]
        h2 = _bdot(a2, xb2c) + z2s.astype(f32)
        h2 = _bdot(h2, t2t_ref[...]) + t2b_ref[...]
        v2 = _leaky(h2 * m2s_ref[...] + m2b_ref[...])

        pieces = []
        for j in range(4):
            acc = jax.lax.slice(v1, (64 * j, 0), (64 * j + 16, _HID))
            for k in range(1, 4):
                acc = acc + jax.lax.slice(
                    v1, (64 * j + 16 * k, 0), (64 * j + 16 * k + 16, _HID))
            pieces.append(acc * 0.25)
        for j in range(3):
            acc = jax.lax.slice(v2, (128 * j, 0), (128 * j + 16, _HID))
            for k in range(1, 8):
                acc = acc + jax.lax.slice(
                    v2, (128 * j + 16 * k, 0), (128 * j + 16 * k + 16, _HID))
            pieces.append(acc * 0.125)
        hcat = jnp.concatenate(pieces, axis=0)

        c = _bdot(hcat.T, wpq_ref[...])
        f = jax.lax.slice(c, (0, 0), (1, 16))
        for hh in range(1, _HID):
            f = f + jax.lax.slice(c, (hh, 16 * hh), (hh + 1, 16 * hh + 16))
        fs.append(f)

    f = jnp.concatenate(fs, axis=0)
    f = jnp.maximum(f + fb1_ref[...], 0.0)
    f = jnp.maximum(_bdot(f, f2t_ref[...]) + fb2_ref[...], 0.0)
    f = jnp.maximum(_bdot(f, f3t_ref[...]) + fb3_ref[...], 0.0)
    y = (jnp.sum(f.astype(bf16).astype(f32)
                 * f4_ref[...].astype(bf16).astype(f32), axis=-1)
         + fb4_ref[0, 0])
    out_ref[...] = jnp.broadcast_to(y[:, None, None], (_G, 8, 128))


@jax.jit
def kernel(X, params):
    p = params
    f32 = jnp.float32
    bs = X.shape[0]

    xu = jnp.transpose(X.reshape(bs, _TLEN, 16, _NN), (0, 1, 3, 2))
    xu = xu.reshape(bs * _TLEN * _NN, 16)

    ti = jnp.arange(16)[:, None] - jnp.arange(16)[None, :]
    bands = jnp.stack([(ti == k - 1).astype(f32) for k in range(3)])
    m1 = jnp.einsum('ck,ktu->ctu', p['conv1_w'][:, 0, :], bands)
    m1 = jnp.transpose(m1, (1, 0, 2)).reshape(16, 128)
    s1v = jnp.repeat(p['bn_c1_g'] / jnp.sqrt(1.0 + _EPS), 16)[None, :]
    b1v = jnp.repeat(p['bn_c1_b'], 16)[None, :]
    m2 = jnp.einsum('oik,ktu->itou', p['conv2_w'], bands).reshape(128, 128)
    s2v = jnp.repeat(p['bn_c2_g'] / jnp.sqrt(1.0 + _EPS), 16)[None, :]
    b2v = jnp.repeat(p['bn_c2_b'], 16)[None, :]

    l2 = p['lin2_w'].T
    l2b = p['lin2_b'][None, :]
    slv = (p['bn2_g'] / jnp.sqrt(1.0 + _EPS))[None, :]
    blb = p['bn2_b'][None, :]

    pos = jnp.arange(_TLEN, dtype=f32)[:, None]
    div = jnp.exp(jnp.arange(0, _D2, 2, dtype=f32) * (-np.log(10000.0) / _D2))
    pe = jnp.zeros((_TLEN, _D2), f32)
    pe = pe.at[:, 0::2].set(jnp.sin(pos * div))
    pe = pe.at[:, 1::2].set(jnp.cos(pos * div))
    pe_exp = jnp.tile(jnp.repeat(pe, _NN, axis=0), (_G, 1))

    def bn_pair(g, b):
        return (g / jnp.sqrt(1.0 + _EPS))[None, :], b[None, :]

    sb1, bb1 = bn_pair(p['bnb1_g'], p['bnb1_b'])
    sb2, bb2 = bn_pair(p['bnb2_g'], p['bnb2_b'])
    m1s, m1b = bn_pair(p['bnm1_g'], p['bnm1_b'])
    m2s, m2b = bn_pair(p['bnm2_g'], p['bnm2_b'])

    r = jnp.arange(_R)
    win1 = r // 64
    inw1 = (win1[:, None] == win1[None, :]).astype(f32)
    eye = jnp.eye(_R, dtype=f32)
    pat = r // 16
    dec = _DECAY ** jnp.abs(pat[:, None] - pat[None, :]).astype(f32)
    mx1 = dec * inw1 - eye
    iw1 = inw1 - eye

    RR = jnp.arange(384)
    j2 = RR // 128
    rl = RR % 128
    col = jnp.arange(_R)[None, :]
    diagcol = (64 * j2 + rl)[:, None]
    inw2 = ((col >= (64 * j2)[:, None]) & (col < (64 * j2 + 128)[:, None]))
    pat_c = (col - (64 * j2)[:, None]) // 16
    dec2 = _DECAY ** jnp.abs((rl // 16)[:, None] - pat_c).astype(f32)
    ond = (col == diagcol)
    mx2 = jnp.where(inw2 & ~ond, dec2, 0.0).astype(f32)
    iw2 = jnp.where(inw2 & ~ond, 1.0, 0.0).astype(f32)

    wpq = jnp.transpose(p['fc1_w'].reshape(_D2, 7 * _NN, _HID),
                        (1, 2, 0)).reshape(7 * _NN, 128)
    fb1 = p['fc1_b'][None, :]
    f2t, fb2 = p['fc2_w'].T, p['fc2_b'][None, :]
    f3t, fb3 = p['fc3_w'].T, p['fc3_b'][None, :]
    f4 = p['fc4_w'][0][None, :]
    fb4 = p['fc4_b'][None, :]

    full = lambda shp: pl.BlockSpec(shp, lambda b: tuple(0 for _ in shp))
    in_specs = [
        pl.BlockSpec((_G * _R, 16), lambda b: (b, 0)),
        full((_G * _R, 16)),
        full((16, 128)), full((1, 128)), full((1, 128)),
        full((128, 128)), full((1, 128)), full((1, 128)),
        full((128, 16)), full((1, 16)), full((1, 16)), full((1, 16)),
        full((16, 16)), full((1, 16)),
        full((16, 16)), full((1, 16)),
        full((1, 16)), full((1, 16)),
        full((1, 16)), full((1, 16)),
        full((16, 8)), full((1, 8)),
        full((1, 8)), full((1, 8)),
        full((16, 8)), full((1, 8)),
        full((1, 8)), full((1, 8)),
        full((_R, _R)), full((_R, _R)),
        full((384, _R)), full((384, _R)),
        full((7 * _NN, 128)), full((1, 16)),
        full((16, 16)), full((1, 16)),
        full((16, 8)), full((1, 8)),
        full((1, 8)), full((1, 1)),
    ]
    out = pl.pallas_call(
        _body,
        grid=(bs // _G,),
        in_specs=in_specs,
        out_specs=pl.BlockSpec((_G, 8, 128), lambda b: (b, 0, 0)),
        out_shape=jax.ShapeDtypeStruct((bs, 8, 128), f32),
        compiler_params=pltpu.CompilerParams(
            dimension_semantics=("arbitrary",),
        ),
    )(xu, pe_exp, m1, s1v, b1v, m2, s2v, b2v, l2, l2b, slv, blb,
      p['g1_w'].T, p['g1_b'][None, :],
      p['g2_w'].T, p['g2_b'][None, :],
      sb1, bb1, sb2, bb2,
      p['t1_w'].T, p['t1_b'][None, :], m1s, m1b,
      p['t2_w'].T, p['t2_b'][None, :], m2s, m2b,
      mx1, iw1, mx2, iw2,
      wpq, fb1, f2t, fb2, f3t, fb3, f4, fb4)
    return out[:, 0, :1]
